# baseline (speedup 1.0000x reference)
import functools

import jax
import jax.numpy as jnp
from jax import lax
from jax.experimental import pallas as pl
from jax.experimental.pallas import tpu as pltpu
from jax.experimental.pallas import tpu_sc as plsc

N = 10000
E = 320000
D = 128
NC = 2
NS = 16
NW = NC * NS
EPT = E // NW
CH = 128
NF = EPT // CH
TAIL = EPT - NF * CH
CHD = 80
NCHD = EPT // CHD
RPT = 640
RPT_LAST = N - RPT * (NS - 1)

_MESH = plsc.VectorSubcoreMesh(core_axis_name="c", subcore_axis_name="s")


@functools.partial(
    pl.kernel,
    out_type=jax.ShapeDtypeStruct((NC * N,), jnp.float32),
    mesh=_MESH,
    scratch_types=[
        pltpu.VMEM((NCHD, CHD), jnp.int32),
        pltpu.VMEM((CHD,), jnp.float32),
        pltpu.VMEM((RPT,), jnp.float32),
        pltpu.VMEM_SHARED((N,), jnp.float32),
        pltpu.SemaphoreType.DMA,
        pltpu.SemaphoreType.DMA,
    ],
)
def _sc_deg(dst3_hbm, out_hbm, idx2_v, ones_v, zrow_v, acc_sh, dsem, ssem):
    c = lax.axis_index("c")
    s = lax.axis_index("s")
    wid = s * NC + c
    zeros16 = jnp.zeros((16,), jnp.float32)
    ones16 = jnp.ones((16,), jnp.float32)
    pltpu.async_copy(dst3_hbm.at[wid], idx2_v, dsem)

    def fill_body(i, _):
        ones_v[pl.ds(i * 16, 16)] = ones16
        return 0

    lax.fori_loop(0, CHD // 16, fill_body, 0)

    def zfill_body(i, _):
        zrow_v[pl.ds(i * 16, 16)] = zeros16
        return 0

    lax.fori_loop(0, RPT // 16, zfill_body, 0)

    @pl.when(s < NS - 1)
    def _():
        pltpu.sync_copy(zrow_v, acc_sh.at[pl.ds(s * RPT, RPT)])

    @pl.when(s == NS - 1)
    def _():
        pltpu.sync_copy(zrow_v.at[pl.ds(0, RPT_LAST)],
                        acc_sh.at[pl.ds((NS - 1) * RPT, RPT_LAST)])

    plsc.subcore_barrier()
    pltpu.make_async_copy(dst3_hbm.at[wid], idx2_v, dsem).wait()

    def fire(g, _):
        pltpu.async_copy(ones_v, acc_sh.at[idx2_v.at[g]], ssem, add=True)
        return 0

    lax.fori_loop(0, NCHD, fire, 0)

    def drain(g, _):
        pltpu.make_async_copy(ones_v, acc_sh.at[idx2_v.at[g]], ssem).wait()
        return 0

    lax.fori_loop(0, NCHD, drain, 0)
    plsc.subcore_barrier()

    @pl.when(s < NS - 1)
    def _():
        pltpu.sync_copy(acc_sh.at[pl.ds(s * RPT, RPT)], zrow_v)
        pltpu.sync_copy(zrow_v, out_hbm.at[pl.ds(c * N + s * RPT, RPT)])

    @pl.when(s == NS - 1)
    def _():
        pltpu.sync_copy(acc_sh.at[pl.ds((NS - 1) * RPT, RPT_LAST)],
                        zrow_v.at[pl.ds(0, RPT_LAST)])
        pltpu.sync_copy(zrow_v.at[pl.ds(0, RPT_LAST)],
                        out_hbm.at[pl.ds(c * N + (NS - 1) * RPT, RPT_LAST)])


@functools.partial(
    pl.kernel,
    out_type=jax.ShapeDtypeStruct((NC, N, D), jnp.float32),
    mesh=_MESH,
    scratch_types=[
        pltpu.VMEM((EPT,), jnp.int32),
        pltpu.VMEM((NCHD, CHD), jnp.int32),
        [pltpu.VMEM((CHD, D), jnp.float32) for _ in range(2)],
        pltpu.VMEM_SHARED((N, D), jnp.float32),
        [pltpu.SemaphoreType.DMA for _ in range(2)],
        pltpu.SemaphoreType.DMA,
        [pltpu.SemaphoreType.DMA for _ in range(2)],
    ],
)
def _sc_hop(src_hbm, dst3_hbm, y_hbm, z_hbm, out_hbm,
            sidx1, didx2, bufs, acc_sh, gsem, dsem, ssem):
    c = lax.axis_index("c")
    s = lax.axis_index("s")
    wid = s * NC + c
    ebase = wid * EPT
    pltpu.async_copy(dst3_hbm.at[wid], didx2, dsem)
    pltpu.sync_copy(src_hbm.at[pl.ds(ebase, EPT)], sidx1)

    def fire_gather(g, slot):
        pltpu.async_copy(y_hbm.at[sidx1.at[pl.ds(g * CHD, CHD)]],
                         bufs[slot], gsem[slot])

    def wait_gather(g, slot):
        pltpu.make_async_copy(y_hbm.at[sidx1.at[pl.ds(g * CHD, CHD)]],
                              bufs[slot], gsem[slot]).wait()

    def wait_scatter(g, slot):
        pltpu.make_async_copy(bufs[slot], acc_sh.at[didx2.at[g]],
                              ssem[slot]).wait()

    fire_gather(0, 0)

    def init_slice(src):
        @pl.when(s < NS - 1)
        def _():
            pltpu.sync_copy(src.at[pl.ds(s * RPT, RPT)],
                            acc_sh.at[pl.ds(s * RPT, RPT)])

        @pl.when(s == NS - 1)
        def _():
            pltpu.sync_copy(src.at[pl.ds((NS - 1) * RPT, RPT_LAST)],
                            acc_sh.at[pl.ds((NS - 1) * RPT, RPT_LAST)])

    @pl.when(c == 0)
    def _():
        init_slice(y_hbm)

    @pl.when(c == 1)
    def _():
        init_slice(z_hbm)

    pltpu.make_async_copy(dst3_hbm.at[wid], didx2, dsem).wait()
    plsc.subcore_barrier()

    def chunk(g, _):
        def step(slot, other):
            @pl.when(g < NCHD - 1)
            def _():
                @pl.when(g >= 1)
                def _():
                    wait_scatter(g - 1, other)

                fire_gather(g + 1, other)

            wait_gather(g, slot)
            pltpu.async_copy(bufs[slot], acc_sh.at[didx2.at[g]],
                             ssem[slot], add=True)

        @pl.when(g % 2 == 0)
        def _():
            step(0, 1)

        @pl.when(g % 2 == 1)
        def _():
            step(1, 0)

        return 0

    lax.fori_loop(0, NCHD, chunk, 0)
    wait_scatter(NCHD - 2, 1)
    wait_scatter(NCHD - 1, 0)
    plsc.subcore_barrier()

    @pl.when(s < NS - 1)
    def _():
        pltpu.sync_copy(acc_sh.at[pl.ds(s * RPT, RPT)],
                        out_hbm.at[c, pl.ds(s * RPT, RPT)])

    @pl.when(s == NS - 1)
    def _():
        pltpu.sync_copy(acc_sh.at[pl.ds((NS - 1) * RPT, RPT_LAST)],
                        out_hbm.at[c, pl.ds((NS - 1) * RPT, RPT_LAST)])


_RB = 1000
_GRID = N // _RB


def _tc_prep_body(h_ref, x_ref, y_ref, d_ref):
    deg = jnp.transpose(1.0 + h_ref[0:1, :] + h_ref[1:2, :])
    d_ref[...] = deg
    y_ref[...] = x_ref[...] * lax.rsqrt(deg)


def _tc_prep(hist, x):
    rb = 1280
    return pl.pallas_call(
        _tc_prep_body,
        grid=(pl.cdiv(N, rb),),
        in_specs=[
            pl.BlockSpec((NC, rb), lambda i: (0, i)),
            pl.BlockSpec((rb, D), lambda i: (i, 0)),
        ],
        out_specs=[
            pl.BlockSpec((rb, D), lambda i: (i, 0)),
            pl.BlockSpec((rb, 1), lambda i: (i, 0)),
        ],
        out_shape=[
            jax.ShapeDtypeStruct((N, D), jnp.float32),
            jax.ShapeDtypeStruct((N, 1), jnp.float32),
        ],
    )(hist, x)


def _tc_combine_body(p_ref, d_ref, o_ref):
    o_ref[...] = (p_ref[0] + p_ref[1]) / d_ref[...]


def _tc_combine(parts, deg):
    return pl.pallas_call(
        _tc_combine_body,
        grid=(_GRID,),
        in_specs=[
            pl.BlockSpec((NC, _RB, D), lambda i: (0, i, 0)),
            pl.BlockSpec((_RB, 1), lambda i: (i, 0)),
        ],
        out_specs=pl.BlockSpec((_RB, D), lambda i: (i, 0)),
        out_shape=jax.ShapeDtypeStruct((N, D), jnp.float32),
    )(parts, deg)


def _tc_final_body(p_ref, d_ref, wt_ref, b_ref, o_ref):
    h = (p_ref[0] + p_ref[1]) * lax.rsqrt(d_ref[...])
    z = jnp.dot(h, wt_ref[...], preferred_element_type=jnp.float32)
    o_ref[...] = jnp.maximum(z + b_ref[...], 0.0)


def _tc_final(parts, deg, wt, brow):
    return pl.pallas_call(
        _tc_final_body,
        grid=(_GRID,),
        in_specs=[
            pl.BlockSpec((NC, _RB, D), lambda i: (0, i, 0)),
            pl.BlockSpec((_RB, 1), lambda i: (i, 0)),
            pl.BlockSpec((D, D), lambda i: (0, 0)),
            pl.BlockSpec((1, D), lambda i: (0, 0)),
        ],
        out_specs=pl.BlockSpec((_RB, D), lambda i: (i, 0)),
        out_shape=jax.ShapeDtypeStruct((N, D), jnp.float32),
    )(parts, deg, wt, brow)


def kernel(x, edge_index, W, b):
    src = edge_index[0]
    dst3 = edge_index[1].reshape(NW, NCHD, CHD)
    hist = _sc_deg(dst3).reshape(NC, N)
    y0, deg = _tc_prep(hist, x)
    zeros = jnp.zeros((N, D), jnp.float32)
    p1 = _sc_hop(src, dst3, y0, zeros)
    y1 = _tc_combine(p1, deg)
    p2 = _sc_hop(src, dst3, y1, zeros)
    return _tc_final(p2, deg, W.T, b.reshape(1, D))


# speedup vs baseline: 1.0798x; 1.0798x over previous
import functools

import jax
import jax.numpy as jnp
from jax import lax
from jax.experimental import pallas as pl
from jax.experimental.pallas import tpu as pltpu
from jax.experimental.pallas import tpu_sc as plsc

N = 10000
E = 320000
D = 128
NC = 2
NS = 16
NW = NC * NS
EPT = E // NW
CH = 128
NF = EPT // CH
TAIL = EPT - NF * CH
CHD = 80
NCHD = EPT // CHD
RPT = 640
RPT_LAST = N - RPT * (NS - 1)

_MESH = plsc.VectorSubcoreMesh(core_axis_name="c", subcore_axis_name="s")


@functools.partial(
    pl.kernel,
    out_type=jax.ShapeDtypeStruct((NC * N,), jnp.float32),
    mesh=_MESH,
    scratch_types=[
        pltpu.VMEM((NCHD, CHD), jnp.int32),
        pltpu.VMEM((CHD,), jnp.float32),
        pltpu.VMEM((RPT,), jnp.float32),
        pltpu.VMEM_SHARED((N,), jnp.float32),
        pltpu.SemaphoreType.DMA,
        pltpu.SemaphoreType.DMA,
    ],
)
def _sc_deg(dst3_hbm, out_hbm, idx2_v, ones_v, zrow_v, acc_sh, dsem, ssem):
    c = lax.axis_index("c")
    s = lax.axis_index("s")
    wid = s * NC + c
    zeros16 = jnp.zeros((16,), jnp.float32)
    ones16 = jnp.ones((16,), jnp.float32)
    pltpu.async_copy(dst3_hbm.at[wid], idx2_v, dsem)

    def fill_body(i, _):
        ones_v[pl.ds(i * 16, 16)] = ones16
        return 0

    lax.fori_loop(0, CHD // 16, fill_body, 0)

    def zfill_body(i, _):
        zrow_v[pl.ds(i * 16, 16)] = zeros16
        return 0

    lax.fori_loop(0, RPT // 16, zfill_body, 0)

    @pl.when(s < NS - 1)
    def _():
        pltpu.sync_copy(zrow_v, acc_sh.at[pl.ds(s * RPT, RPT)])

    @pl.when(s == NS - 1)
    def _():
        pltpu.sync_copy(zrow_v.at[pl.ds(0, RPT_LAST)],
                        acc_sh.at[pl.ds((NS - 1) * RPT, RPT_LAST)])

    plsc.subcore_barrier()
    pltpu.make_async_copy(dst3_hbm.at[wid], idx2_v, dsem).wait()

    def fire(g, _):
        pltpu.async_copy(ones_v, acc_sh.at[idx2_v.at[g]], ssem, add=True)
        return 0

    lax.fori_loop(0, NCHD, fire, 0)

    def drain(g, _):
        pltpu.make_async_copy(ones_v, acc_sh.at[idx2_v.at[g]], ssem).wait()
        return 0

    lax.fori_loop(0, NCHD, drain, 0)
    plsc.subcore_barrier()

    @pl.when(s < NS - 1)
    def _():
        pltpu.sync_copy(acc_sh.at[pl.ds(s * RPT, RPT)], zrow_v)
        pltpu.sync_copy(zrow_v, out_hbm.at[pl.ds(c * N + s * RPT, RPT)])

    @pl.when(s == NS - 1)
    def _():
        pltpu.sync_copy(acc_sh.at[pl.ds((NS - 1) * RPT, RPT_LAST)],
                        zrow_v.at[pl.ds(0, RPT_LAST)])
        pltpu.sync_copy(zrow_v.at[pl.ds(0, RPT_LAST)],
                        out_hbm.at[pl.ds(c * N + (NS - 1) * RPT, RPT_LAST)])


@functools.partial(
    pl.kernel,
    out_type=jax.ShapeDtypeStruct((NC, N, D), jnp.float32),
    mesh=_MESH,
    scratch_types=[
        pltpu.VMEM((EPT,), jnp.int32),
        [pltpu.VMEM((CH,), jnp.int32) for _ in range(2)],
        pltpu.VMEM((TAIL,), jnp.int32),
        [pltpu.VMEM((CH, D), jnp.float32) for _ in range(2)],
        pltpu.VMEM((TAIL, D), jnp.float32),
        pltpu.VMEM_SHARED((N, D), jnp.float32),
        [pltpu.SemaphoreType.DMA for _ in range(2)],
        [pltpu.SemaphoreType.DMA for _ in range(2)],
        [pltpu.SemaphoreType.DMA for _ in range(2)],
    ],
)
def _sc_hop(src_hbm, dst_hbm, y_hbm, z_hbm, out_hbm,
            sidx1, didx, tidx_v, bufs, tbuf, acc_sh, gsem, dsem, ssem):
    c = lax.axis_index("c")
    s = lax.axis_index("s")
    wid = s * NC + c
    ebase = wid * EPT
    pltpu.sync_copy(src_hbm.at[pl.ds(ebase, EPT)], sidx1)

    def load_idx(g, slot):
        pltpu.async_copy(dst_hbm.at[pl.ds(ebase + g * CH, CH)],
                         didx[slot], dsem[slot])

    def fire_gather(g, slot):
        pltpu.async_copy(y_hbm.at[sidx1.at[pl.ds(g * CH, CH)]],
                         bufs[slot], gsem[slot])

    def wait_idx(g, slot):
        pltpu.make_async_copy(dst_hbm.at[pl.ds(ebase + g * CH, CH)],
                              didx[slot], dsem[slot]).wait()

    def wait_gather(g, slot):
        pltpu.make_async_copy(y_hbm.at[sidx1.at[pl.ds(g * CH, CH)]],
                              bufs[slot], gsem[slot]).wait()

    def wait_scatter(slot):
        pltpu.make_async_copy(bufs[slot], acc_sh.at[didx[slot]],
                              ssem[slot]).wait()

    load_idx(0, 0)
    fire_gather(0, 0)

    def init_slice(src):
        @pl.when(s < NS - 1)
        def _():
            pltpu.sync_copy(src.at[pl.ds(s * RPT, RPT)],
                            acc_sh.at[pl.ds(s * RPT, RPT)])

        @pl.when(s == NS - 1)
        def _():
            pltpu.sync_copy(src.at[pl.ds((NS - 1) * RPT, RPT_LAST)],
                            acc_sh.at[pl.ds((NS - 1) * RPT, RPT_LAST)])

    @pl.when(c == 0)
    def _():
        init_slice(y_hbm)

    @pl.when(c == 1)
    def _():
        init_slice(z_hbm)

    plsc.subcore_barrier()

    def chunk(g, _):
        def step(slot, other):
            @pl.when(g < NF - 1)
            def _():
                @pl.when(g >= 1)
                def _():
                    wait_scatter(other)

                load_idx(g + 1, other)
                fire_gather(g + 1, other)

            wait_idx(g, slot)
            wait_gather(g, slot)
            pltpu.async_copy(bufs[slot], acc_sh.at[didx[slot]],
                             ssem[slot], add=True)

        @pl.when(g % 2 == 0)
        def _():
            step(0, 1)

        @pl.when(g % 2 == 1)
        def _():
            step(1, 0)

        return 0

    lax.fori_loop(0, NF, chunk, 0)
    wait_scatter(0)
    wait_scatter(1)
    pltpu.sync_copy(dst_hbm.at[pl.ds(ebase + NF * CH, TAIL)], tidx_v)
    pltpu.async_copy(y_hbm.at[sidx1.at[pl.ds(NF * CH, TAIL)]],
                     tbuf, gsem[0]).wait()
    pltpu.sync_copy(tbuf, acc_sh.at[tidx_v], add=True)
    plsc.subcore_barrier()

    @pl.when(s < NS - 1)
    def _():
        pltpu.sync_copy(acc_sh.at[pl.ds(s * RPT, RPT)],
                        out_hbm.at[c, pl.ds(s * RPT, RPT)])

    @pl.when(s == NS - 1)
    def _():
        pltpu.sync_copy(acc_sh.at[pl.ds((NS - 1) * RPT, RPT_LAST)],
                        out_hbm.at[c, pl.ds((NS - 1) * RPT, RPT_LAST)])


_RB = 1000
_GRID = N // _RB


def _tc_prep_body(h_ref, x_ref, y_ref, d_ref):
    deg = jnp.transpose(1.0 + h_ref[0:1, :] + h_ref[1:2, :])
    d_ref[...] = deg
    y_ref[...] = x_ref[...] * lax.rsqrt(deg)


def _tc_prep(hist, x):
    rb = 1280
    return pl.pallas_call(
        _tc_prep_body,
        grid=(pl.cdiv(N, rb),),
        in_specs=[
            pl.BlockSpec((NC, rb), lambda i: (0, i)),
            pl.BlockSpec((rb, D), lambda i: (i, 0)),
        ],
        out_specs=[
            pl.BlockSpec((rb, D), lambda i: (i, 0)),
            pl.BlockSpec((rb, 1), lambda i: (i, 0)),
        ],
        out_shape=[
            jax.ShapeDtypeStruct((N, D), jnp.float32),
            jax.ShapeDtypeStruct((N, 1), jnp.float32),
        ],
    )(hist, x)


def _tc_combine_body(p_ref, d_ref, o_ref):
    o_ref[...] = (p_ref[0] + p_ref[1]) / d_ref[...]


def _tc_combine(parts, deg):
    return pl.pallas_call(
        _tc_combine_body,
        grid=(_GRID,),
        in_specs=[
            pl.BlockSpec((NC, _RB, D), lambda i: (0, i, 0)),
            pl.BlockSpec((_RB, 1), lambda i: (i, 0)),
        ],
        out_specs=pl.BlockSpec((_RB, D), lambda i: (i, 0)),
        out_shape=jax.ShapeDtypeStruct((N, D), jnp.float32),
    )(parts, deg)


def _tc_final_body(p_ref, d_ref, wt_ref, b_ref, o_ref):
    h = (p_ref[0] + p_ref[1]) * lax.rsqrt(d_ref[...])
    z = jnp.dot(h, wt_ref[...], preferred_element_type=jnp.float32)
    o_ref[...] = jnp.maximum(z + b_ref[...], 0.0)


def _tc_final(parts, deg, wt, brow):
    return pl.pallas_call(
        _tc_final_body,
        grid=(_GRID,),
        in_specs=[
            pl.BlockSpec((NC, _RB, D), lambda i: (0, i, 0)),
            pl.BlockSpec((_RB, 1), lambda i: (i, 0)),
            pl.BlockSpec((D, D), lambda i: (0, 0)),
            pl.BlockSpec((1, D), lambda i: (0, 0)),
        ],
        out_specs=pl.BlockSpec((_RB, D), lambda i: (i, 0)),
        out_shape=jax.ShapeDtypeStruct((N, D), jnp.float32),
    )(parts, deg, wt, brow)


def kernel(x, edge_index, W, b):
    src = edge_index[0]
    dst = edge_index[1]
    hist = _sc_deg(dst.reshape(NW, NCHD, CHD)).reshape(NC, N)
    y0, deg = _tc_prep(hist, x)
    zeros = jnp.zeros((N, D), jnp.float32)
    p1 = _sc_hop(src, dst, y0, zeros)
    y1 = _tc_combine(p1, deg)
    p2 = _sc_hop(src, dst, y1, zeros)
    return _tc_final(p2, deg, W.T, b.reshape(1, D))


# speedup vs baseline: 1.1388x; 1.0546x over previous
import functools

import jax
import jax.numpy as jnp
from jax import lax
from jax.experimental import pallas as pl
from jax.experimental.pallas import tpu as pltpu
from jax.experimental.pallas import tpu_sc as plsc

N = 10000
E = 320000
D = 128
NC = 2
NS = 16
NW = NC * NS
EPT = E // NW
CH = 128
NF = EPT // CH
TAIL = EPT - NF * CH
CHD = 80
NCHD = EPT // CHD
RPT = 640
RPT_LAST = N - RPT * (NS - 1)

_MESH = plsc.VectorSubcoreMesh(core_axis_name="c", subcore_axis_name="s")


@functools.partial(
    pl.kernel,
    out_type=jax.ShapeDtypeStruct((NC * N,), jnp.float32),
    mesh=_MESH,
    scratch_types=[
        pltpu.VMEM((NCHD, CHD), jnp.int32),
        pltpu.VMEM((CHD,), jnp.float32),
        pltpu.VMEM((RPT,), jnp.float32),
        pltpu.VMEM_SHARED((N,), jnp.float32),
        pltpu.SemaphoreType.DMA,
        pltpu.SemaphoreType.DMA,
    ],
)
def _sc_deg(dst3_hbm, out_hbm, idx2_v, ones_v, zrow_v, acc_sh, dsem, ssem):
    c = lax.axis_index("c")
    s = lax.axis_index("s")
    wid = s * NC + c
    zeros16 = jnp.zeros((16,), jnp.float32)
    ones16 = jnp.ones((16,), jnp.float32)
    pltpu.async_copy(dst3_hbm.at[wid], idx2_v, dsem)

    def fill_body(i, _):
        ones_v[pl.ds(i * 16, 16)] = ones16
        return 0

    lax.fori_loop(0, CHD // 16, fill_body, 0)

    def zfill_body(i, _):
        zrow_v[pl.ds(i * 16, 16)] = zeros16
        return 0

    lax.fori_loop(0, RPT // 16, zfill_body, 0)

    @pl.when(s < NS - 1)
    def _():
        pltpu.sync_copy(zrow_v, acc_sh.at[pl.ds(s * RPT, RPT)])

    @pl.when(s == NS - 1)
    def _():
        pltpu.sync_copy(zrow_v.at[pl.ds(0, RPT_LAST)],
                        acc_sh.at[pl.ds((NS - 1) * RPT, RPT_LAST)])

    plsc.subcore_barrier()
    pltpu.make_async_copy(dst3_hbm.at[wid], idx2_v, dsem).wait()

    def fire(g, _):
        pltpu.async_copy(ones_v, acc_sh.at[idx2_v.at[g]], ssem, add=True)
        return 0

    lax.fori_loop(0, NCHD, fire, 0)

    def drain(g, _):
        pltpu.make_async_copy(ones_v, acc_sh.at[idx2_v.at[g]], ssem).wait()
        return 0

    lax.fori_loop(0, NCHD, drain, 0)
    plsc.subcore_barrier()

    @pl.when(s < NS - 1)
    def _():
        pltpu.sync_copy(acc_sh.at[pl.ds(s * RPT, RPT)], zrow_v)
        pltpu.sync_copy(zrow_v, out_hbm.at[pl.ds(c * N + s * RPT, RPT)])

    @pl.when(s == NS - 1)
    def _():
        pltpu.sync_copy(acc_sh.at[pl.ds((NS - 1) * RPT, RPT_LAST)],
                        zrow_v.at[pl.ds(0, RPT_LAST)])
        pltpu.sync_copy(zrow_v.at[pl.ds(0, RPT_LAST)],
                        out_hbm.at[pl.ds(c * N + (NS - 1) * RPT, RPT_LAST)])


@functools.partial(
    pl.kernel,
    out_type=jax.ShapeDtypeStruct((NC, N, D), jnp.float32),
    mesh=_MESH,
    scratch_types=[
        pltpu.VMEM((EPT,), jnp.int32),
        [pltpu.VMEM((CH,), jnp.int32) for _ in range(2)],
        pltpu.VMEM((TAIL,), jnp.int32),
        [pltpu.VMEM((CH, D), jnp.float32) for _ in range(2)],
        pltpu.VMEM((TAIL, D), jnp.float32),
        pltpu.VMEM_SHARED((N, D), jnp.float32),
        [pltpu.SemaphoreType.DMA for _ in range(2)],
        [pltpu.SemaphoreType.DMA for _ in range(2)],
        [pltpu.SemaphoreType.DMA for _ in range(2)],
    ],
)
def _sc_hop(src_hbm, dst_hbm, y_hbm, z_hbm, out_hbm,
            sidx1, didx, tidx_v, bufs, tbuf, acc_sh, gsem, dsem, ssem):
    c = lax.axis_index("c")
    s = lax.axis_index("s")
    wid = s * NC + c
    ebase = wid * EPT
    pltpu.sync_copy(src_hbm.at[pl.ds(ebase, EPT)], sidx1)

    def load_idx(g, slot):
        pltpu.async_copy(dst_hbm.at[pl.ds(ebase + g * CH, CH)],
                         didx[slot], dsem[slot])

    def fire_gather(g, slot):
        pltpu.async_copy(y_hbm.at[sidx1.at[pl.ds(g * CH, CH)]],
                         bufs[slot], gsem[slot])

    def wait_idx(g, slot):
        pltpu.make_async_copy(dst_hbm.at[pl.ds(ebase + g * CH, CH)],
                              didx[slot], dsem[slot]).wait()

    def wait_gather(g, slot):
        pltpu.make_async_copy(y_hbm.at[sidx1.at[pl.ds(g * CH, CH)]],
                              bufs[slot], gsem[slot]).wait()

    def wait_scatter(slot):
        pltpu.make_async_copy(bufs[slot], acc_sh.at[didx[slot]],
                              ssem[slot]).wait()

    load_idx(0, 0)
    fire_gather(0, 0)

    def init_slice(src):
        @pl.when(s < NS - 1)
        def _():
            pltpu.sync_copy(src.at[pl.ds(s * RPT, RPT)],
                            acc_sh.at[pl.ds(s * RPT, RPT)])

        @pl.when(s == NS - 1)
        def _():
            pltpu.sync_copy(src.at[pl.ds((NS - 1) * RPT, RPT_LAST)],
                            acc_sh.at[pl.ds((NS - 1) * RPT, RPT_LAST)])

    @pl.when(c == 0)
    def _():
        init_slice(y_hbm)

    @pl.when(c == 1)
    def _():
        init_slice(z_hbm)

    plsc.subcore_barrier()

    def chunk(g, _):
        def step(slot, other):
            @pl.when(g < NF - 1)
            def _():
                @pl.when(g >= 1)
                def _():
                    wait_scatter(other)

                load_idx(g + 1, other)
                fire_gather(g + 1, other)

            wait_idx(g, slot)
            wait_gather(g, slot)
            pltpu.async_copy(bufs[slot], acc_sh.at[didx[slot]],
                             ssem[slot], add=True)

        @pl.when(g % 2 == 0)
        def _():
            step(0, 1)

        @pl.when(g % 2 == 1)
        def _():
            step(1, 0)

        return 0

    lax.fori_loop(0, NF, chunk, 0)
    wait_scatter(0)
    wait_scatter(1)
    pltpu.sync_copy(dst_hbm.at[pl.ds(ebase + NF * CH, TAIL)], tidx_v)
    pltpu.async_copy(y_hbm.at[sidx1.at[pl.ds(NF * CH, TAIL)]],
                     tbuf, gsem[0]).wait()
    pltpu.sync_copy(tbuf, acc_sh.at[tidx_v], add=True)
    plsc.subcore_barrier()

    @pl.when(s < NS - 1)
    def _():
        pltpu.sync_copy(acc_sh.at[pl.ds(s * RPT, RPT)],
                        out_hbm.at[c, pl.ds(s * RPT, RPT)])

    @pl.when(s == NS - 1)
    def _():
        pltpu.sync_copy(acc_sh.at[pl.ds((NS - 1) * RPT, RPT_LAST)],
                        out_hbm.at[c, pl.ds((NS - 1) * RPT, RPT_LAST)])


_RB = 2000
_GRID = N // _RB


def _tc_split_body(e_ref, s_ref, d_ref):
    s_ref[...] = e_ref[0, :]
    d_ref[...] = e_ref[1, :]


def _tc_split(edge_index):
    cb = 32768
    return pl.pallas_call(
        _tc_split_body,
        grid=(pl.cdiv(E, cb),),
        in_specs=[pl.BlockSpec((2, cb), lambda i: (0, i))],
        out_specs=[
            pl.BlockSpec((cb,), lambda i: (i,)),
            pl.BlockSpec((cb,), lambda i: (i,)),
        ],
        out_shape=[
            jax.ShapeDtypeStruct((E,), jnp.int32),
            jax.ShapeDtypeStruct((E,), jnp.int32),
        ],
    )(edge_index)


def _tc_prep_body(h_ref, x_ref, y_ref, d_ref):
    deg = jnp.transpose(1.0 + h_ref[0:1, :] + h_ref[1:2, :])
    d_ref[...] = deg
    y_ref[...] = x_ref[...] * lax.rsqrt(deg)


def _tc_prep(hist, x):
    rb = 2560
    return pl.pallas_call(
        _tc_prep_body,
        grid=(pl.cdiv(N, rb),),
        in_specs=[
            pl.BlockSpec((NC, rb), lambda i: (0, i)),
            pl.BlockSpec((rb, D), lambda i: (i, 0)),
        ],
        out_specs=[
            pl.BlockSpec((rb, D), lambda i: (i, 0)),
            pl.BlockSpec((rb, 1), lambda i: (i, 0)),
        ],
        out_shape=[
            jax.ShapeDtypeStruct((N, D), jnp.float32),
            jax.ShapeDtypeStruct((N, 1), jnp.float32),
        ],
    )(hist, x)


def _tc_combine_body(p_ref, d_ref, o_ref):
    o_ref[...] = (p_ref[0] + p_ref[1]) / d_ref[...]


def _tc_combine(parts, deg):
    return pl.pallas_call(
        _tc_combine_body,
        grid=(_GRID,),
        in_specs=[
            pl.BlockSpec((NC, _RB, D), lambda i: (0, i, 0)),
            pl.BlockSpec((_RB, 1), lambda i: (i, 0)),
        ],
        out_specs=pl.BlockSpec((_RB, D), lambda i: (i, 0)),
        out_shape=jax.ShapeDtypeStruct((N, D), jnp.float32),
    )(parts, deg)


def _tc_final_body(p_ref, d_ref, wt_ref, b_ref, o_ref):
    h = (p_ref[0] + p_ref[1]) * lax.rsqrt(d_ref[...])
    z = jnp.dot(h, wt_ref[...], preferred_element_type=jnp.float32)
    o_ref[...] = jnp.maximum(z + b_ref[...], 0.0)


def _tc_final(parts, deg, wt, brow):
    return pl.pallas_call(
        _tc_final_body,
        grid=(_GRID,),
        in_specs=[
            pl.BlockSpec((NC, _RB, D), lambda i: (0, i, 0)),
            pl.BlockSpec((_RB, 1), lambda i: (i, 0)),
            pl.BlockSpec((D, D), lambda i: (0, 0)),
            pl.BlockSpec((1, D), lambda i: (0, 0)),
        ],
        out_specs=pl.BlockSpec((_RB, D), lambda i: (i, 0)),
        out_shape=jax.ShapeDtypeStruct((N, D), jnp.float32),
    )(parts, deg, wt, brow)


def kernel(x, edge_index, W, b):
    src, dst = _tc_split(edge_index)
    hist = _sc_deg(dst.reshape(NW, NCHD, CHD)).reshape(NC, N)
    y0, deg = _tc_prep(hist, x)
    zeros = jnp.zeros((N, D), jnp.float32)
    p1 = _sc_hop(src, dst, y0, zeros)
    y1 = _tc_combine(p1, deg)
    p2 = _sc_hop(src, dst, y1, zeros)
    return _tc_final(p2, deg, W.T, b.reshape(1, D))


# speedup vs baseline: 1.1587x; 1.0175x over previous
import functools

import jax
import jax.numpy as jnp
from jax import lax
from jax.experimental import pallas as pl
from jax.experimental.pallas import tpu as pltpu
from jax.experimental.pallas import tpu_sc as plsc

N = 10000
E = 320000
D = 128
NC = 2
NS = 16
NW = NC * NS
EPT = E // NW
CH = 128
NF = EPT // CH
TAIL = EPT - NF * CH
CHD = 80
NCHD = EPT // CHD
RPT = 640
RPT_LAST = N - RPT * (NS - 1)

_MESH = plsc.VectorSubcoreMesh(core_axis_name="c", subcore_axis_name="s")


@functools.partial(
    pl.kernel,
    out_type=jax.ShapeDtypeStruct((NC * N,), jnp.float32),
    mesh=_MESH,
    scratch_types=[
        pltpu.VMEM((EPT,), jnp.int32),
        pltpu.VMEM((CHD,), jnp.float32),
        pltpu.VMEM((RPT,), jnp.float32),
        pltpu.VMEM_SHARED((N,), jnp.float32),
        pltpu.SemaphoreType.DMA,
        pltpu.SemaphoreType.DMA,
    ],
)
def _sc_deg(dst_hbm, out_hbm, idx1_v, ones_v, zrow_v, acc_sh, dsem, ssem):
    c = lax.axis_index("c")
    s = lax.axis_index("s")
    wid = s * NC + c
    zeros16 = jnp.zeros((16,), jnp.float32)
    ones16 = jnp.ones((16,), jnp.float32)
    pltpu.async_copy(dst_hbm.at[pl.ds(wid * EPT, EPT)], idx1_v, dsem)

    def fill_body(i, _):
        ones_v[pl.ds(i * 16, 16)] = ones16
        return 0

    lax.fori_loop(0, CHD // 16, fill_body, 0)

    def zfill_body(i, _):
        zrow_v[pl.ds(i * 16, 16)] = zeros16
        return 0

    lax.fori_loop(0, RPT // 16, zfill_body, 0)

    @pl.when(s < NS - 1)
    def _():
        pltpu.sync_copy(zrow_v, acc_sh.at[pl.ds(s * RPT, RPT)])

    @pl.when(s == NS - 1)
    def _():
        pltpu.sync_copy(zrow_v.at[pl.ds(0, RPT_LAST)],
                        acc_sh.at[pl.ds((NS - 1) * RPT, RPT_LAST)])

    plsc.subcore_barrier()
    pltpu.make_async_copy(dst_hbm.at[pl.ds(wid * EPT, EPT)], idx1_v,
                          dsem).wait()

    def fire(g, _):
        pltpu.async_copy(ones_v, acc_sh.at[idx1_v.at[pl.ds(g * CHD, CHD)]],
                         ssem, add=True)
        return 0

    lax.fori_loop(0, NCHD, fire, 0)

    def drain(g, _):
        pltpu.make_async_copy(ones_v,
                              acc_sh.at[idx1_v.at[pl.ds(g * CHD, CHD)]],
                              ssem).wait()
        return 0

    lax.fori_loop(0, NCHD, drain, 0)
    plsc.subcore_barrier()

    @pl.when(s < NS - 1)
    def _():
        pltpu.sync_copy(acc_sh.at[pl.ds(s * RPT, RPT)], zrow_v)
        pltpu.sync_copy(zrow_v, out_hbm.at[pl.ds(c * N + s * RPT, RPT)])

    @pl.when(s == NS - 1)
    def _():
        pltpu.sync_copy(acc_sh.at[pl.ds((NS - 1) * RPT, RPT_LAST)],
                        zrow_v.at[pl.ds(0, RPT_LAST)])
        pltpu.sync_copy(zrow_v.at[pl.ds(0, RPT_LAST)],
                        out_hbm.at[pl.ds(c * N + (NS - 1) * RPT, RPT_LAST)])


@functools.partial(
    pl.kernel,
    out_type=jax.ShapeDtypeStruct((NC, N, D), jnp.float32),
    mesh=_MESH,
    scratch_types=[
        pltpu.VMEM((EPT,), jnp.int32),
        [pltpu.VMEM((CH,), jnp.int32) for _ in range(2)],
        pltpu.VMEM((TAIL,), jnp.int32),
        [pltpu.VMEM((CH, D), jnp.float32) for _ in range(2)],
        pltpu.VMEM((TAIL, D), jnp.float32),
        pltpu.VMEM_SHARED((N, D), jnp.float32),
        [pltpu.SemaphoreType.DMA for _ in range(2)],
        [pltpu.SemaphoreType.DMA for _ in range(2)],
        [pltpu.SemaphoreType.DMA for _ in range(2)],
    ],
)
def _sc_hop(src_hbm, dst_hbm, y_hbm, z_hbm, out_hbm,
            sidx1, didx, tidx_v, bufs, tbuf, acc_sh, gsem, dsem, ssem):
    c = lax.axis_index("c")
    s = lax.axis_index("s")
    wid = s * NC + c
    ebase = wid * EPT
    pltpu.sync_copy(src_hbm.at[pl.ds(ebase, EPT)], sidx1)

    def load_idx(g, slot):
        pltpu.async_copy(dst_hbm.at[pl.ds(ebase + g * CH, CH)],
                         didx[slot], dsem[slot])

    def fire_gather(g, slot):
        pltpu.async_copy(y_hbm.at[sidx1.at[pl.ds(g * CH, CH)]],
                         bufs[slot], gsem[slot])

    def wait_idx(g, slot):
        pltpu.make_async_copy(dst_hbm.at[pl.ds(ebase + g * CH, CH)],
                              didx[slot], dsem[slot]).wait()

    def wait_gather(g, slot):
        pltpu.make_async_copy(y_hbm.at[sidx1.at[pl.ds(g * CH, CH)]],
                              bufs[slot], gsem[slot]).wait()

    def wait_scatter(slot):
        pltpu.make_async_copy(bufs[slot], acc_sh.at[didx[slot]],
                              ssem[slot]).wait()

    load_idx(0, 0)
    fire_gather(0, 0)

    def init_slice(src):
        @pl.when(s < NS - 1)
        def _():
            pltpu.sync_copy(src.at[pl.ds(s * RPT, RPT)],
                            acc_sh.at[pl.ds(s * RPT, RPT)])

        @pl.when(s == NS - 1)
        def _():
            pltpu.sync_copy(src.at[pl.ds((NS - 1) * RPT, RPT_LAST)],
                            acc_sh.at[pl.ds((NS - 1) * RPT, RPT_LAST)])

    @pl.when(c == 0)
    def _():
        init_slice(y_hbm)

    @pl.when(c == 1)
    def _():
        init_slice(z_hbm)

    plsc.subcore_barrier()

    def chunk(g, _):
        def step(slot, other):
            @pl.when(g < NF - 1)
            def _():
                @pl.when(g >= 1)
                def _():
                    wait_scatter(other)

                load_idx(g + 1, other)
                fire_gather(g + 1, other)

            wait_idx(g, slot)
            wait_gather(g, slot)
            pltpu.async_copy(bufs[slot], acc_sh.at[didx[slot]],
                             ssem[slot], add=True)

        @pl.when(g % 2 == 0)
        def _():
            step(0, 1)

        @pl.when(g % 2 == 1)
        def _():
            step(1, 0)

        return 0

    lax.fori_loop(0, NF, chunk, 0)
    wait_scatter(0)
    wait_scatter(1)
    pltpu.sync_copy(dst_hbm.at[pl.ds(ebase + NF * CH, TAIL)], tidx_v)
    pltpu.async_copy(y_hbm.at[sidx1.at[pl.ds(NF * CH, TAIL)]],
                     tbuf, gsem[0]).wait()
    pltpu.sync_copy(tbuf, acc_sh.at[tidx_v], add=True)
    plsc.subcore_barrier()

    @pl.when(s < NS - 1)
    def _():
        pltpu.sync_copy(acc_sh.at[pl.ds(s * RPT, RPT)],
                        out_hbm.at[c, pl.ds(s * RPT, RPT)])

    @pl.when(s == NS - 1)
    def _():
        pltpu.sync_copy(acc_sh.at[pl.ds((NS - 1) * RPT, RPT_LAST)],
                        out_hbm.at[c, pl.ds((NS - 1) * RPT, RPT_LAST)])


_RB = 2000
_GRID = N // _RB


def _tc_split_body(e_ref, s_ref, d_ref):
    s_ref[...] = e_ref[0, :]
    d_ref[...] = e_ref[1, :]


def _tc_split(edge_index):
    cb = 32768
    return pl.pallas_call(
        _tc_split_body,
        grid=(pl.cdiv(E, cb),),
        in_specs=[pl.BlockSpec((2, cb), lambda i: (0, i))],
        out_specs=[
            pl.BlockSpec((cb,), lambda i: (i,)),
            pl.BlockSpec((cb,), lambda i: (i,)),
        ],
        out_shape=[
            jax.ShapeDtypeStruct((E,), jnp.int32),
            jax.ShapeDtypeStruct((E,), jnp.int32),
        ],
    )(edge_index)


def _tc_prep_body(h_ref, x_ref, y_ref, d_ref):
    deg = jnp.transpose(1.0 + h_ref[0:1, :] + h_ref[1:2, :])
    d_ref[...] = deg
    y_ref[...] = x_ref[...] * lax.rsqrt(deg)


def _tc_prep(hist, x):
    rb = 2560
    return pl.pallas_call(
        _tc_prep_body,
        grid=(pl.cdiv(N, rb),),
        in_specs=[
            pl.BlockSpec((NC, rb), lambda i: (0, i)),
            pl.BlockSpec((rb, D), lambda i: (i, 0)),
        ],
        out_specs=[
            pl.BlockSpec((rb, D), lambda i: (i, 0)),
            pl.BlockSpec((rb, 1), lambda i: (i, 0)),
        ],
        out_shape=[
            jax.ShapeDtypeStruct((N, D), jnp.float32),
            jax.ShapeDtypeStruct((N, 1), jnp.float32),
        ],
    )(hist, x)


def _tc_combine_body(p_ref, d_ref, o_ref):
    o_ref[...] = (p_ref[0] + p_ref[1]) / d_ref[...]


def _tc_combine(parts, deg):
    return pl.pallas_call(
        _tc_combine_body,
        grid=(_GRID,),
        in_specs=[
            pl.BlockSpec((NC, _RB, D), lambda i: (0, i, 0)),
            pl.BlockSpec((_RB, 1), lambda i: (i, 0)),
        ],
        out_specs=pl.BlockSpec((_RB, D), lambda i: (i, 0)),
        out_shape=jax.ShapeDtypeStruct((N, D), jnp.float32),
    )(parts, deg)


def _tc_final_body(p_ref, d_ref, wt_ref, b_ref, o_ref):
    h = (p_ref[0] + p_ref[1]) * lax.rsqrt(d_ref[...])
    z = jnp.dot(h, wt_ref[...], preferred_element_type=jnp.float32)
    o_ref[...] = jnp.maximum(z + b_ref[...], 0.0)


def _tc_final(parts, deg, wt, brow):
    return pl.pallas_call(
        _tc_final_body,
        grid=(_GRID,),
        in_specs=[
            pl.BlockSpec((NC, _RB, D), lambda i: (0, i, 0)),
            pl.BlockSpec((_RB, 1), lambda i: (i, 0)),
            pl.BlockSpec((D, D), lambda i: (0, 0)),
            pl.BlockSpec((1, D), lambda i: (0, 0)),
        ],
        out_specs=pl.BlockSpec((_RB, D), lambda i: (i, 0)),
        out_shape=jax.ShapeDtypeStruct((N, D), jnp.float32),
    )(parts, deg, wt, brow)


def kernel(x, edge_index, W, b):
    src, dst = _tc_split(edge_index)
    hist = _sc_deg(dst).reshape(NC, N)
    y0, deg = _tc_prep(hist, x)
    zeros = jnp.zeros((N, D), jnp.float32)
    p1 = _sc_hop(src, dst, y0, zeros)
    y1 = _tc_combine(p1, deg)
    p2 = _sc_hop(src, dst, y1, zeros)
    return _tc_final(p2, deg, W.T, b.reshape(1, D))


# speedup vs baseline: 1.1712x; 1.0108x over previous
import functools

import jax
import jax.numpy as jnp
from jax import lax
from jax.experimental import pallas as pl
from jax.experimental.pallas import tpu as pltpu
from jax.experimental.pallas import tpu_sc as plsc

N = 10000
E = 320000
D = 128
NC = 2
NS = 16
NW = NC * NS
EPT = E // NW
CH = 128
NF = EPT // CH
TAIL = EPT - NF * CH
CHD = 80
NCHD = EPT // CHD
RPT = 640
RPT_LAST = N - RPT * (NS - 1)

_MESH = plsc.VectorSubcoreMesh(core_axis_name="c", subcore_axis_name="s")


@functools.partial(
    pl.kernel,
    out_type=jax.ShapeDtypeStruct((NC * N,), jnp.float32),
    mesh=_MESH,
    scratch_types=[
        pltpu.VMEM((EPT,), jnp.int32),
        pltpu.VMEM((CHD,), jnp.float32),
        pltpu.VMEM((RPT,), jnp.float32),
        pltpu.VMEM_SHARED((N,), jnp.float32),
        pltpu.SemaphoreType.DMA,
        pltpu.SemaphoreType.DMA,
    ],
)
def _sc_deg(dst_hbm, out_hbm, idx1_v, ones_v, zrow_v, acc_sh, dsem, ssem):
    c = lax.axis_index("c")
    s = lax.axis_index("s")
    wid = s * NC + c
    zeros16 = jnp.zeros((16,), jnp.float32)
    ones16 = jnp.ones((16,), jnp.float32)
    pltpu.async_copy(dst_hbm.at[pl.ds(wid * EPT, EPT)], idx1_v, dsem)

    def fill_body(i, _):
        ones_v[pl.ds(i * 16, 16)] = ones16
        return 0

    lax.fori_loop(0, CHD // 16, fill_body, 0)

    def zfill_body(i, _):
        zrow_v[pl.ds(i * 16, 16)] = zeros16
        return 0

    lax.fori_loop(0, RPT // 16, zfill_body, 0)

    @pl.when(s < NS - 1)
    def _():
        pltpu.sync_copy(zrow_v, acc_sh.at[pl.ds(s * RPT, RPT)])

    @pl.when(s == NS - 1)
    def _():
        pltpu.sync_copy(zrow_v.at[pl.ds(0, RPT_LAST)],
                        acc_sh.at[pl.ds((NS - 1) * RPT, RPT_LAST)])

    plsc.subcore_barrier()
    pltpu.make_async_copy(dst_hbm.at[pl.ds(wid * EPT, EPT)], idx1_v,
                          dsem).wait()

    def fire(g, _):
        pltpu.async_copy(ones_v, acc_sh.at[idx1_v.at[pl.ds(g * CHD, CHD)]],
                         ssem, add=True)
        return 0

    lax.fori_loop(0, NCHD, fire, 0)

    def drain(g, _):
        pltpu.make_async_copy(ones_v,
                              acc_sh.at[idx1_v.at[pl.ds(g * CHD, CHD)]],
                              ssem).wait()
        return 0

    lax.fori_loop(0, NCHD, drain, 0)
    plsc.subcore_barrier()

    @pl.when(s < NS - 1)
    def _():
        pltpu.sync_copy(acc_sh.at[pl.ds(s * RPT, RPT)], zrow_v)
        pltpu.sync_copy(zrow_v, out_hbm.at[pl.ds(c * N + s * RPT, RPT)])

    @pl.when(s == NS - 1)
    def _():
        pltpu.sync_copy(acc_sh.at[pl.ds((NS - 1) * RPT, RPT_LAST)],
                        zrow_v.at[pl.ds(0, RPT_LAST)])
        pltpu.sync_copy(zrow_v.at[pl.ds(0, RPT_LAST)],
                        out_hbm.at[pl.ds(c * N + (NS - 1) * RPT, RPT_LAST)])


@functools.partial(
    pl.kernel,
    out_type=jax.ShapeDtypeStruct((NC, N, D), jnp.float32),
    mesh=_MESH,
    scratch_types=[
        pltpu.VMEM((EPT,), jnp.int32),
        [pltpu.VMEM((CH,), jnp.int32) for _ in range(2)],
        pltpu.VMEM((TAIL,), jnp.int32),
        [pltpu.VMEM((CH, D), jnp.float32) for _ in range(2)],
        pltpu.VMEM((TAIL, D), jnp.float32),
        pltpu.VMEM((32, D), jnp.float32),
        pltpu.VMEM_SHARED((N, D), jnp.float32),
        [pltpu.SemaphoreType.DMA for _ in range(2)],
        [pltpu.SemaphoreType.DMA for _ in range(2)],
        [pltpu.SemaphoreType.DMA for _ in range(2)],
    ],
)
def _sc_hop(src_hbm, dst_hbm, y_hbm, out_hbm,
            sidx1, didx, tidx_v, bufs, tbuf, zbuf, acc_sh, gsem, dsem, ssem):
    c = lax.axis_index("c")
    s = lax.axis_index("s")
    wid = s * NC + c
    ebase = wid * EPT
    pltpu.sync_copy(src_hbm.at[pl.ds(ebase, EPT)], sidx1)

    def load_idx(g, slot):
        pltpu.async_copy(dst_hbm.at[pl.ds(ebase + g * CH, CH)],
                         didx[slot], dsem[slot])

    def fire_gather(g, slot):
        pltpu.async_copy(y_hbm.at[sidx1.at[pl.ds(g * CH, CH)]],
                         bufs[slot], gsem[slot])

    def wait_idx(g, slot):
        pltpu.make_async_copy(dst_hbm.at[pl.ds(ebase + g * CH, CH)],
                              didx[slot], dsem[slot]).wait()

    def wait_gather(g, slot):
        pltpu.make_async_copy(y_hbm.at[sidx1.at[pl.ds(g * CH, CH)]],
                              bufs[slot], gsem[slot]).wait()

    def wait_scatter(slot):
        pltpu.make_async_copy(bufs[slot], acc_sh.at[didx[slot]],
                              ssem[slot]).wait()

    load_idx(0, 0)
    fire_gather(0, 0)

    @pl.when(c == 0)
    def _():
        @pl.when(s < NS - 1)
        def _():
            pltpu.sync_copy(y_hbm.at[pl.ds(s * RPT, RPT)],
                            acc_sh.at[pl.ds(s * RPT, RPT)])

        @pl.when(s == NS - 1)
        def _():
            pltpu.sync_copy(y_hbm.at[pl.ds((NS - 1) * RPT, RPT_LAST)],
                            acc_sh.at[pl.ds((NS - 1) * RPT, RPT_LAST)])

    @pl.when(c == 1)
    def _():
        zeros16 = jnp.zeros((16,), jnp.float32)

        def zfill(k, _):
            zbuf[k // 8, pl.ds((k % 8) * 16, 16)] = zeros16
            return 0

        lax.fori_loop(0, 32 * 8, zfill, 0)

        def zcopy(i, _):
            pltpu.sync_copy(zbuf, acc_sh.at[pl.ds(s * RPT + i * 32, 32)])
            return 0

        @pl.when(s < NS - 1)
        def _():
            lax.fori_loop(0, RPT // 32, zcopy, 0)

        @pl.when(s == NS - 1)
        def _():
            lax.fori_loop(0, RPT_LAST // 32, zcopy, 0)
            pltpu.sync_copy(
                zbuf.at[pl.ds(0, RPT_LAST % 32)],
                acc_sh.at[pl.ds(s * RPT + (RPT_LAST // 32) * 32,
                                RPT_LAST % 32)])

    plsc.subcore_barrier()

    def chunk(g, _):
        def step(slot, other):
            @pl.when(g < NF - 1)
            def _():
                @pl.when(g >= 1)
                def _():
                    wait_scatter(other)

                load_idx(g + 1, other)
                fire_gather(g + 1, other)

            wait_idx(g, slot)
            wait_gather(g, slot)
            pltpu.async_copy(bufs[slot], acc_sh.at[didx[slot]],
                             ssem[slot], add=True)

        @pl.when(g % 2 == 0)
        def _():
            step(0, 1)

        @pl.when(g % 2 == 1)
        def _():
            step(1, 0)

        return 0

    lax.fori_loop(0, NF, chunk, 0)
    wait_scatter(0)
    wait_scatter(1)
    pltpu.sync_copy(dst_hbm.at[pl.ds(ebase + NF * CH, TAIL)], tidx_v)
    pltpu.async_copy(y_hbm.at[sidx1.at[pl.ds(NF * CH, TAIL)]],
                     tbuf, gsem[0]).wait()
    pltpu.sync_copy(tbuf, acc_sh.at[tidx_v], add=True)
    plsc.subcore_barrier()

    @pl.when(s < NS - 1)
    def _():
        pltpu.sync_copy(acc_sh.at[pl.ds(s * RPT, RPT)],
                        out_hbm.at[c, pl.ds(s * RPT, RPT)])

    @pl.when(s == NS - 1)
    def _():
        pltpu.sync_copy(acc_sh.at[pl.ds((NS - 1) * RPT, RPT_LAST)],
                        out_hbm.at[c, pl.ds((NS - 1) * RPT, RPT_LAST)])


_RB = 2000
_GRID = N // _RB


def _tc_split_body(e_ref, s_ref, d_ref):
    s_ref[...] = e_ref[0, :]
    d_ref[...] = e_ref[1, :]


def _tc_split(edge_index):
    cb = 32768
    return pl.pallas_call(
        _tc_split_body,
        grid=(pl.cdiv(E, cb),),
        in_specs=[pl.BlockSpec((2, cb), lambda i: (0, i))],
        out_specs=[
            pl.BlockSpec((cb,), lambda i: (i,)),
            pl.BlockSpec((cb,), lambda i: (i,)),
        ],
        out_shape=[
            jax.ShapeDtypeStruct((E,), jnp.int32),
            jax.ShapeDtypeStruct((E,), jnp.int32),
        ],
    )(edge_index)


def _tc_prep_body(h_ref, x_ref, y_ref, d_ref):
    deg = jnp.transpose(1.0 + h_ref[0:1, :] + h_ref[1:2, :])
    d_ref[...] = deg
    y_ref[...] = x_ref[...] * lax.rsqrt(deg)


def _tc_prep(hist, x):
    rb = 2560
    return pl.pallas_call(
        _tc_prep_body,
        grid=(pl.cdiv(N, rb),),
        in_specs=[
            pl.BlockSpec((NC, rb), lambda i: (0, i)),
            pl.BlockSpec((rb, D), lambda i: (i, 0)),
        ],
        out_specs=[
            pl.BlockSpec((rb, D), lambda i: (i, 0)),
            pl.BlockSpec((rb, 1), lambda i: (i, 0)),
        ],
        out_shape=[
            jax.ShapeDtypeStruct((N, D), jnp.float32),
            jax.ShapeDtypeStruct((N, 1), jnp.float32),
        ],
    )(hist, x)


def _tc_combine_body(p_ref, d_ref, o_ref):
    o_ref[...] = (p_ref[0] + p_ref[1]) / d_ref[...]


def _tc_combine(parts, deg):
    return pl.pallas_call(
        _tc_combine_body,
        grid=(_GRID,),
        in_specs=[
            pl.BlockSpec((NC, _RB, D), lambda i: (0, i, 0)),
            pl.BlockSpec((_RB, 1), lambda i: (i, 0)),
        ],
        out_specs=pl.BlockSpec((_RB, D), lambda i: (i, 0)),
        out_shape=jax.ShapeDtypeStruct((N, D), jnp.float32),
    )(parts, deg)


def _tc_final_body(p_ref, d_ref, wt_ref, b_ref, o_ref):
    h = (p_ref[0] + p_ref[1]) * lax.rsqrt(d_ref[...])
    z = jnp.dot(h, wt_ref[...], preferred_element_type=jnp.float32)
    o_ref[...] = jnp.maximum(z + b_ref[...], 0.0)


def _tc_final(parts, deg, wt, brow):
    return pl.pallas_call(
        _tc_final_body,
        grid=(_GRID,),
        in_specs=[
            pl.BlockSpec((NC, _RB, D), lambda i: (0, i, 0)),
            pl.BlockSpec((_RB, 1), lambda i: (i, 0)),
            pl.BlockSpec((D, D), lambda i: (0, 0)),
            pl.BlockSpec((1, D), lambda i: (0, 0)),
        ],
        out_specs=pl.BlockSpec((_RB, D), lambda i: (i, 0)),
        out_shape=jax.ShapeDtypeStruct((N, D), jnp.float32),
    )(parts, deg, wt, brow)


def kernel(x, edge_index, W, b):
    src, dst = _tc_split(edge_index)
    hist = _sc_deg(dst).reshape(NC, N)
    y0, deg = _tc_prep(hist, x)
    p1 = _sc_hop(src, dst, y0)
    y1 = _tc_combine(p1, deg)
    p2 = _sc_hop(src, dst, y1)
    return _tc_final(p2, deg, W.T, b.reshape(1, D))
